# baseline (speedup 1.0000x reference)
import functools

import jax
import jax.numpy as jnp
from jax import lax
from jax.experimental import pallas as pl
from jax.experimental.pallas import tpu as pltpu
from jax.experimental.pallas import tpu_sc as plsc

_BATCH = 256
_RES = 65536
_DIM_OUT = 1024
_M = _RES // _DIM_OUT
_S0 = 30976
_S1 = _RES - _S0
_RING = _RES + _S0
_NGRP = _DIM_OUT // 32


def _make_group_loop(idx_v, ring_v, out_v, odd):
    def do_group(kb, base0):
        def gather_m(m, accs):
            acc0, acc1 = accs
            word = idx_v[pl.ds(m * 512 + kb * 16, 16)]
            i0 = word & 0xFFFF
            i1 = lax.shift_right_logical(word, 16)
            if odd:
                f0 = lax.shift_right_arithmetic(i0 - _S1, 31)
                f1 = lax.shift_right_arithmetic(i1 - _S1, 31)
                i0 = i0 + (~f0 & _S0)
                i1 = i1 + (~f1 & _S0)
            acc0 = acc0 + plsc.load_gather(ring_v, [i0])
            acc1 = acc1 + plsc.load_gather(ring_v, [i1])
            return acc0, acc1

        zero = jnp.zeros((16,), jnp.float32)
        acc0, acc1 = lax.fori_loop(0, _M, gather_m, (zero, zero),
                                   unroll=8 if odd else 16)
        off = (kb - base0) * 32
        out_v[pl.ds(off, 16)] = jnp.where(acc0 > 0.5, 1.0, acc0)
        out_v[pl.ds(off + 16, 16)] = jnp.where(acc1 > 0.5, 1.0, acc1)
        return base0

    return do_group


def _readout_body(x_hbm, idx_hbm, out_hbm, idx_v, ring_v, out_v,
                  idx_sem, pf_sem):
    info = plsc.get_sparse_core_info()
    nc = info.num_cores
    nw = nc * info.num_subcores
    rows_per_w = _BATCH // nw
    wid = lax.axis_index("s") * nc + lax.axis_index("c")
    row0 = wid * rows_per_w

    idx_cp = pltpu.async_copy(idx_hbm, idx_v, idx_sem)
    pf0 = pltpu.async_copy(x_hbm.at[row0, pl.ds(0, _S0)],
                           ring_v.at[pl.ds(_S1, _S0)], pf_sem)
    pltpu.sync_copy(x_hbm.at[row0, pl.ds(_S0, _S1)],
                    ring_v.at[pl.ds(0, _S1)])
    pf0.wait()
    idx_cp.wait()

    def do_row(r, _):
        row = row0 + r
        nxt = row0 + jnp.minimum(r + 1, rows_per_w - 1)
        pf = pltpu.async_copy(
            x_hbm.at[nxt, pl.ds(0, _S0)],
            ring_v.at[pl.ds(_S1 + ((r + 1) % 2) * _S0, _S0)], pf_sem)

        def run_row(odd):
            do_group = _make_group_loop(idx_v, ring_v, out_v, odd)
            lax.fori_loop(0, _NGRP // 2, do_group, 0)
            pltpu.sync_copy(out_v, out_hbm.at[row, pl.ds(0, 512)])
            lax.fori_loop(_NGRP // 2, _NGRP, do_group, _NGRP // 2)
            pltpu.sync_copy(out_v, out_hbm.at[row, pl.ds(512, 512)])

        @pl.when(r % 2 == 0)
        def _():
            run_row(odd=False)

        @pl.when(r % 2 == 1)
        def _():
            run_row(odd=True)

        pf.wait()

        @pl.when(r < rows_per_w - 1)
        def _():
            pltpu.sync_copy(x_hbm.at[nxt, pl.ds(_S0, _S1)],
                            ring_v.at[pl.ds(0, _S1)])

        return 0

    lax.fori_loop(0, rows_per_w, do_row, 0)


@jax.jit
def _readout(x, idx_packed):
    mesh = plsc.VectorSubcoreMesh(core_axis_name="c", subcore_axis_name="s")
    k = functools.partial(
        pl.kernel,
        mesh=mesh,
        out_type=jax.ShapeDtypeStruct((_BATCH, _DIM_OUT), jnp.float32),
        scratch_types=[
            pltpu.VMEM((_RES // 2,), jnp.int32),
            pltpu.VMEM((_RING,), jnp.float32),
            pltpu.VMEM((512,), jnp.float32),
            pltpu.SemaphoreType.DMA,
            pltpu.SemaphoreType.DMA,
        ],
        compiler_params=pltpu.CompilerParams(needs_layout_passes=False),
    )(_readout_body)
    return k(x, idx_packed)


def kernel(x, pre, post):
    del post
    w = jnp.where(pre < _S0, pre + _S1, pre - _S0)
    p = w.reshape(_M, _NGRP, 2, 16)
    packed = p[:, :, 0, :] | (p[:, :, 1, :] << 16)
    return _readout(x, packed.reshape(-1))


# speedup vs baseline: 1.0885x; 1.0885x over previous
import functools

import jax
import jax.numpy as jnp
from jax import lax
from jax.experimental import pallas as pl
from jax.experimental.pallas import tpu as pltpu
from jax.experimental.pallas import tpu_sc as plsc

_BATCH = 256
_RES = 65536
_DIM_OUT = 1024
_M = _RES // _DIM_OUT
_HALF = _DIM_OUT // 2
_ROWS_PER_W = 8


def _readout_body(x_hbm, idx_hbm, out_hbm, idx_v, row_v, out_a, out_b,
                  idx_sem, row_sem_a, row_sem_b, out_sem_a, out_sem_b):
    info = plsc.get_sparse_core_info()
    nc = info.num_cores
    wid = lax.axis_index("s") * nc + lax.axis_index("c")
    row0 = wid * _ROWS_PER_W

    idx_cp = pltpu.async_copy(idx_hbm, idx_v, idx_sem)

    def stage_row(row):
        h = _RES // 2
        ca = pltpu.async_copy(x_hbm.at[row, pl.ds(0, h)],
                              row_v.at[pl.ds(0, h)], row_sem_a)
        cb = pltpu.async_copy(x_hbm.at[row, pl.ds(h, h)],
                              row_v.at[pl.ds(h, h)], row_sem_b)
        return ca, cb

    def do_groups(out_v):
        def do_group(kb, _):
            base = kb * 16

            def gather_m(m, accs):
                acc0, acc1 = accs
                word = idx_v[pl.ds(m * _HALF + base, 16)]
                i0 = word & 0xFFFF
                i1 = lax.shift_right_logical(word, 16)
                acc0 = acc0 + plsc.load_gather(row_v, [i0])
                acc1 = acc1 + plsc.load_gather(row_v, [i1])
                return acc0, acc1

            zero = jnp.zeros((16,), jnp.float32)
            acc0, acc1 = lax.fori_loop(0, _M, gather_m, (zero, zero),
                                       unroll=16)
            out_v[pl.ds(base, 16)] = jnp.where(acc0 > 0.5, 1.0, acc0)
            out_v[pl.ds(_HALF + base, 16)] = jnp.where(acc1 > 0.5, 1.0, acc1)
            return 0

        lax.fori_loop(0, _HALF // 16, do_group, 0)

    ca, cb = stage_row(row0)
    ca.wait()
    cb.wait()
    idx_cp.wait()
    for r in range(_ROWS_PER_W):
        do_groups(out_a)
        pltpu.sync_copy(out_a, out_hbm.at[row0 + r])
        if r + 1 < _ROWS_PER_W:
            ca, cb = stage_row(row0 + r + 1)
            ca.wait()
            cb.wait()


@jax.jit
def _readout(x, idx_packed):
    mesh = plsc.VectorSubcoreMesh(core_axis_name="c", subcore_axis_name="s")
    k = functools.partial(
        pl.kernel,
        mesh=mesh,
        out_type=jax.ShapeDtypeStruct((_BATCH, _DIM_OUT), jnp.float32),
        scratch_types=[
            pltpu.VMEM((_RES // 2,), jnp.int32),
            pltpu.VMEM((_RES,), jnp.float32),
            pltpu.VMEM((_DIM_OUT,), jnp.float32),
            pltpu.VMEM((_DIM_OUT,), jnp.float32),
            pltpu.SemaphoreType.DMA,
            pltpu.SemaphoreType.DMA,
            pltpu.SemaphoreType.DMA,
            pltpu.SemaphoreType.DMA,
            pltpu.SemaphoreType.DMA,
        ],
        compiler_params=pltpu.CompilerParams(needs_layout_passes=False),
    )(_readout_body)
    return k(x, idx_packed)


def kernel(x, pre, post):
    del post
    p = pre.reshape(_M, _DIM_OUT)
    packed = p[:, :_HALF] | (p[:, _HALF:] << 16)
    return _readout(x, packed.reshape(-1))


# speedup vs baseline: 1.1167x; 1.0260x over previous
import functools

import jax
import jax.numpy as jnp
from jax import lax
from jax.experimental import pallas as pl
from jax.experimental.pallas import tpu as pltpu
from jax.experimental.pallas import tpu_sc as plsc

_BATCH = 256
_RES = 65536
_DIM_OUT = 1024
_M = _RES // _DIM_OUT
_HALF = _DIM_OUT // 2


def _readout_body(x_hbm, idx_hbm, out_hbm, idx_v, row_v, out_v):
    info = plsc.get_sparse_core_info()
    nc = info.num_cores
    nw = nc * info.num_subcores
    rows_per_w = _BATCH // nw
    wid = lax.axis_index("s") * nc + lax.axis_index("c")

    pltpu.sync_copy(idx_hbm, idx_v)

    def do_row(r, _):
        row = wid * rows_per_w + r
        pltpu.sync_copy(x_hbm.at[row], row_v)

        def do_group(kb, _):
            base = kb * 16

            def gather_m(mm, accs):
                a0, a1, b0, b1 = accs
                w0 = idx_v[pl.ds((2 * mm) * _HALF + base, 16)]
                w1 = idx_v[pl.ds((2 * mm + 1) * _HALF + base, 16)]
                a0 = a0 + plsc.load_gather(row_v, [w0 & 0xFFFF])
                a1 = a1 + plsc.load_gather(
                    row_v, [lax.shift_right_logical(w0, 16)])
                b0 = b0 + plsc.load_gather(row_v, [w1 & 0xFFFF])
                b1 = b1 + plsc.load_gather(
                    row_v, [lax.shift_right_logical(w1, 16)])
                return a0, a1, b0, b1

            zero = jnp.zeros((16,), jnp.float32)
            a0, a1, b0, b1 = lax.fori_loop(0, _M // 2, gather_m,
                                           (zero, zero, zero, zero),
                                           unroll=8)
            acc0 = a0 + b0
            acc1 = a1 + b1
            out_v[pl.ds(base, 16)] = jnp.where(acc0 > 0.5, 1.0, acc0)
            out_v[pl.ds(_HALF + base, 16)] = jnp.where(acc1 > 0.5, 1.0, acc1)
            return 0

        lax.fori_loop(0, _HALF // 16, do_group, 0)
        pltpu.sync_copy(out_v, out_hbm.at[row])
        return 0

    lax.fori_loop(0, rows_per_w, do_row, 0)


@jax.jit
def _readout(x, idx_packed):
    mesh = plsc.VectorSubcoreMesh(core_axis_name="c", subcore_axis_name="s")
    k = functools.partial(
        pl.kernel,
        mesh=mesh,
        out_type=jax.ShapeDtypeStruct((_BATCH, _DIM_OUT), jnp.float32),
        scratch_types=[
            pltpu.VMEM((_RES // 2,), jnp.int32),
            pltpu.VMEM((_RES,), jnp.float32),
            pltpu.VMEM((_DIM_OUT,), jnp.float32),
        ],
        compiler_params=pltpu.CompilerParams(needs_layout_passes=False),
    )(_readout_body)
    return k(x, idx_packed)


def kernel(x, pre, post):
    del post
    p = pre.reshape(_M, _DIM_OUT)
    packed = p[:, :_HALF] | (p[:, _HALF:] << 16)
    return _readout(x, packed.reshape(-1))
